# Initial kernel scaffold; baseline (speedup 1.0000x reference)
import jax
import jax.numpy as jnp
from jax.experimental import pallas as pl


def kernel(x, gate_w, gate_proj, up_proj, down_proj):
    raise NotImplementedError("write your pallas kernel here")


# speedup vs baseline: 3.6681x; 3.6681x over previous
import jax
import jax.numpy as jnp
from jax.experimental import pallas as pl
from jax.experimental.pallas import tpu as pltpu

_HIDDEN = 2048
_NE = 16
_TOPK = 2
_INTER = 768
_TOKENS = 4096
_ASSIGN = _TOKENS * _TOPK
_TM = 256
_N_TILES = (_ASSIGN + _NE * (_TM - 1) + _TM - 1) // _TM
_M_PAD = _N_TILES * _TM
_ROUTER_BM = 512
_LANE = 128


def _router_body(x_ref, gw_ref, logits_ref, meta_ref):
    logits = jnp.dot(x_ref[...], gw_ref[...], preferred_element_type=jnp.float32)
    cols = jax.lax.broadcasted_iota(jnp.int32, logits.shape, 1)
    neg = jnp.float32(-jnp.inf)
    lm = jnp.where(cols < _NE, logits, neg)
    m0 = jnp.max(lm, axis=1, keepdims=True)
    i0 = jnp.min(jnp.where(lm == m0, cols, _NE), axis=1, keepdims=True)
    lm1 = jnp.where(cols == i0, neg, lm)
    m1 = jnp.max(lm1, axis=1, keepdims=True)
    i1 = jnp.min(jnp.where(lm1 == m1, cols, _NE), axis=1, keepdims=True)
    d = jnp.exp(m1 - m0)
    w0 = 1.0 / (1.0 + d)
    w1 = d / (1.0 + d)
    logits_ref[...] = logits
    meta = jnp.where(cols == 0, w0,
           jnp.where(cols == 1, w1,
           jnp.where(cols == 2, i0.astype(jnp.float32),
           jnp.where(cols == 3, i1.astype(jnp.float32), 0.0))))
    meta_ref[...] = meta


def _moe_body(te_ref, tv_ref, x_ref, g_ref, u_ref, d_ref, out_ref):
    s = pl.program_id(0)

    @pl.when(tv_ref[s] == 1)
    def _():
        x = x_ref[...]
        g = jnp.dot(x, g_ref[0], preferred_element_type=jnp.float32)
        u = jnp.dot(x, u_ref[0], preferred_element_type=jnp.float32)
        sig = 1.0 / (1.0 + jnp.exp(-g))
        act = g * sig * u
        out_ref[...] = jnp.dot(act, d_ref[0], preferred_element_type=jnp.float32)


def kernel(x, gate_w, gate_proj, up_proj, down_proj):
    x2 = x.reshape(-1, _HIDDEN)

    gw_pad = jnp.zeros((_HIDDEN, _LANE), jnp.float32).at[:, :_NE].set(gate_w)
    logits_pad, meta = pl.pallas_call(
        _router_body,
        grid=(_TOKENS // _ROUTER_BM,),
        in_specs=[
            pl.BlockSpec((_ROUTER_BM, _HIDDEN), lambda i: (i, 0)),
            pl.BlockSpec((_HIDDEN, _LANE), lambda i: (0, 0)),
        ],
        out_specs=[
            pl.BlockSpec((_ROUTER_BM, _LANE), lambda i: (i, 0)),
            pl.BlockSpec((_ROUTER_BM, _LANE), lambda i: (i, 0)),
        ],
        out_shape=[
            jax.ShapeDtypeStruct((_TOKENS, _LANE), jnp.float32),
            jax.ShapeDtypeStruct((_TOKENS, _LANE), jnp.float32),
        ],
    )(x2, gw_pad)

    router_logits = logits_pad[:, :_NE]
    rw = meta[:, :_TOPK]
    sel = meta[:, _TOPK:2 * _TOPK].astype(jnp.int32)

    sel_flat = sel.reshape(-1)
    onehot = (sel_flat[:, None] == jnp.arange(_NE)[None, :]).astype(jnp.int32)
    cum = jnp.cumsum(onehot, axis=0)
    counts = cum[-1]
    rank = jnp.take_along_axis(cum, sel_flat[:, None], axis=1)[:, 0] - 1
    padded = ((counts + _TM - 1) // _TM) * _TM
    bounds = jnp.cumsum(padded)
    pstart = bounds - padded
    pos = pstart[sel_flat] + rank

    tgrid = jnp.arange(_N_TILES, dtype=jnp.int32) * _TM
    tile_e = jnp.minimum(
        jnp.searchsorted(bounds, tgrid, side='right').astype(jnp.int32), _NE - 1)
    tile_v = (tgrid < bounds[-1]).astype(jnp.int32)

    tok_of = jnp.arange(_ASSIGN) // _TOPK
    x_sp = jnp.zeros((_M_PAD, _HIDDEN), jnp.float32).at[pos].set(x2[tok_of])

    out_sp = pl.pallas_call(
        _moe_body,
        grid_spec=pltpu.PrefetchScalarGridSpec(
            num_scalar_prefetch=2,
            grid=(_N_TILES,),
            in_specs=[
                pl.BlockSpec((_TM, _HIDDEN), lambda s, te, tv: (s, 0)),
                pl.BlockSpec((1, _HIDDEN, _INTER), lambda s, te, tv: (te[s], 0, 0)),
                pl.BlockSpec((1, _HIDDEN, _INTER), lambda s, te, tv: (te[s], 0, 0)),
                pl.BlockSpec((1, _INTER, _HIDDEN), lambda s, te, tv: (te[s], 0, 0)),
            ],
            out_specs=pl.BlockSpec((_TM, _HIDDEN), lambda s, te, tv: (s, 0)),
        ),
        out_shape=jax.ShapeDtypeStruct((_M_PAD, _HIDDEN), jnp.float32),
        compiler_params=pltpu.CompilerParams(
            dimension_semantics=("arbitrary",),
        ),
    )(tile_e, tile_v, x_sp, gate_proj, up_proj, down_proj)

    pair = out_sp[pos].reshape(_TOKENS, _TOPK, _HIDDEN)
    final = (pair * rw[:, :, None]).sum(axis=1)
    return (final.reshape(x.shape), router_logits)


# speedup vs baseline: 4.2687x; 1.1637x over previous
import jax
import jax.numpy as jnp
from jax.experimental import pallas as pl
from jax.experimental.pallas import tpu as pltpu

_HIDDEN = 2048
_NE = 16
_TOPK = 2
_INTER = 768
_TOKENS = 4096
_ASSIGN = _TOKENS * _TOPK
_TM = 256
_N_TILES = (_ASSIGN + _NE * (_TM - 1) + _TM - 1) // _TM
_M_PAD = _N_TILES * _TM
_ROUTER_BM = 512
_LANE = 128


def _router_body(x_ref, gw_ref, logits_ref, meta_ref):
    logits = jnp.dot(x_ref[...], gw_ref[...], preferred_element_type=jnp.float32)
    cols = jax.lax.broadcasted_iota(jnp.int32, logits.shape, 1)
    neg = jnp.float32(-jnp.inf)
    lm = jnp.where(cols < _NE, logits, neg)
    m0 = jnp.max(lm, axis=1, keepdims=True)
    i0 = jnp.min(jnp.where(lm == m0, cols, _NE), axis=1, keepdims=True)
    lm1 = jnp.where(cols == i0, neg, lm)
    m1 = jnp.max(lm1, axis=1, keepdims=True)
    i1 = jnp.min(jnp.where(lm1 == m1, cols, _NE), axis=1, keepdims=True)
    d = jnp.exp(m1 - m0)
    w0 = 1.0 / (1.0 + d)
    w1 = d / (1.0 + d)
    logits_ref[...] = logits
    meta = jnp.where(cols == 0, w0,
           jnp.where(cols == 1, w1,
           jnp.where(cols == 2, i0.astype(jnp.float32),
           jnp.where(cols == 3, i1.astype(jnp.float32), 0.0))))
    meta_ref[...] = meta


def _moe_body(te_ref, tv_ref, x_ref, g_ref, u_ref, d_ref, w_ref, out_ref):
    s = pl.program_id(0)

    @pl.when(tv_ref[s] == 1)
    def _():
        x = x_ref[...]
        g = jnp.dot(x, g_ref[0], preferred_element_type=jnp.float32)
        u = jnp.dot(x, u_ref[0], preferred_element_type=jnp.float32)
        sig = 1.0 / (1.0 + jnp.exp(-g))
        act = g * sig * u * w_ref[...]
        out_ref[...] = jnp.dot(act, d_ref[0], preferred_element_type=jnp.float32)


def kernel(x, gate_w, gate_proj, up_proj, down_proj):
    x2 = x.reshape(-1, _HIDDEN)

    gw_pad = jnp.zeros((_HIDDEN, _LANE), jnp.float32).at[:, :_NE].set(gate_w)
    logits_pad, meta = pl.pallas_call(
        _router_body,
        grid=(_TOKENS // _ROUTER_BM,),
        in_specs=[
            pl.BlockSpec((_ROUTER_BM, _HIDDEN), lambda i: (i, 0)),
            pl.BlockSpec((_HIDDEN, _LANE), lambda i: (0, 0)),
        ],
        out_specs=[
            pl.BlockSpec((_ROUTER_BM, _LANE), lambda i: (i, 0)),
            pl.BlockSpec((_ROUTER_BM, _LANE), lambda i: (i, 0)),
        ],
        out_shape=[
            jax.ShapeDtypeStruct((_TOKENS, _LANE), jnp.float32),
            jax.ShapeDtypeStruct((_TOKENS, _LANE), jnp.float32),
        ],
    )(x2, gw_pad)

    router_logits = logits_pad[:, :_NE]
    rw = meta[:, :_TOPK]
    sel = meta[:, _TOPK:2 * _TOPK].astype(jnp.int32)

    sel_flat = sel.reshape(-1)
    onehot = (sel_flat[:, None] == jnp.arange(_NE)[None, :]).astype(jnp.int32)
    cum = jnp.cumsum(onehot, axis=0)
    counts = cum[-1]
    rank = jnp.take_along_axis(cum, sel_flat[:, None], axis=1)[:, 0] - 1
    padded = ((counts + _TM - 1) // _TM) * _TM
    bounds = jnp.cumsum(padded)
    pstart = bounds - padded
    pos = pstart[sel_flat] + rank

    tgrid = jnp.arange(_N_TILES, dtype=jnp.int32) * _TM
    tile_e = jnp.minimum(
        jnp.searchsorted(bounds, tgrid, side='right').astype(jnp.int32), _NE - 1)
    tile_v = (tgrid < bounds[-1]).astype(jnp.int32)

    tok_of = (jnp.arange(_ASSIGN) // _TOPK).astype(jnp.int32)
    inv = jnp.zeros((_M_PAD,), jnp.int32).at[pos].set(tok_of)
    x_sp = x2[inv]
    w_pos = jnp.zeros((_M_PAD, 1), jnp.float32).at[pos, 0].set(rw.reshape(-1))

    out_sp = pl.pallas_call(
        _moe_body,
        grid_spec=pltpu.PrefetchScalarGridSpec(
            num_scalar_prefetch=2,
            grid=(_N_TILES,),
            in_specs=[
                pl.BlockSpec((_TM, _HIDDEN), lambda s, te, tv: (s, 0)),
                pl.BlockSpec((1, _HIDDEN, _INTER), lambda s, te, tv: (te[s], 0, 0)),
                pl.BlockSpec((1, _HIDDEN, _INTER), lambda s, te, tv: (te[s], 0, 0)),
                pl.BlockSpec((1, _INTER, _HIDDEN), lambda s, te, tv: (te[s], 0, 0)),
                pl.BlockSpec((_TM, 1), lambda s, te, tv: (s, 0)),
            ],
            out_specs=pl.BlockSpec((_TM, _HIDDEN), lambda s, te, tv: (s, 0)),
        ),
        out_shape=jax.ShapeDtypeStruct((_M_PAD, _HIDDEN), jnp.float32),
        compiler_params=pltpu.CompilerParams(
            dimension_semantics=("arbitrary",),
        ),
    )(tile_e, tile_v, x_sp, gate_proj, up_proj, down_proj, w_pos)

    pair = out_sp[pos].reshape(_TOKENS, _TOPK, _HIDDEN)
    final = pair.sum(axis=1)
    return (final.reshape(x.shape), router_logits)
